# baseline (speedup 1.0000x reference)
import jax
import jax.numpy as jnp
from jax import lax
from jax.experimental import pallas as pl
from jax.experimental.pallas import tpu as pltpu
from jax.experimental.pallas import tpu_sc as plsc

B = 16384
F = 100
D = 16
V = 100000
H1, H2, H3 = 16, 64, 16
NCLS = 2

NC = 2
NS = 16
NW = NC * NS
BPW = B // NW
RING = 16
NOUTER = BPW // RING

_mesh = plsc.VectorSubcoreMesh(
    core_axis_name="c", subcore_axis_name="s", num_cores=NC, num_subcores=NS
)

_SC_SCRATCH = (
    [pltpu.VMEM((BPW, F), jnp.int32)]
    + [pltpu.VMEM((F, D), jnp.float32) for _ in range(RING)]
    + [pltpu.VMEM((BPW, D), jnp.float32)]
    + [pltpu.VMEM((D,), jnp.float32)]
    + [pltpu.SemaphoreType.DMA for _ in range(RING)]
)


def _probs_sc_body(x_hbm, emb2_hbm, bcv_hbm, out_hbm, idx_v, *rest):
    bufs = rest[:RING]
    pooled_v = rest[RING]
    bcv_v = rest[RING + 1]
    sems = rest[RING + 2 : 2 * RING + 2]

    wid = lax.axis_index("s") * NC + lax.axis_index("c")
    row_base = wid * BPW

    pltpu.sync_copy(x_hbm.at[pl.ds(row_base, BPW)], idx_v)
    pltpu.sync_copy(bcv_hbm, bcv_v)

    for r in range(RING):
        pltpu.async_copy(emb2_hbm.at[idx_v.at[r]], bufs[r], sems[r])

    def outer_body(g, _):
        bcv = bcv_v[...]
        for r in range(RING):
            s = g * RING + r
            pltpu.make_async_copy(emb2_hbm.at[idx_v.at[s]], bufs[r], sems[r]).wait()

            rows_v = bufs[r]
            a0 = rows_v[0, :]
            a1 = rows_v[1, :]
            a2 = rows_v[2, :]
            a3 = rows_v[3, :]
            for f in range(4, F, 4):
                a0 = a0 + rows_v[f + 0, :]
                a1 = a1 + rows_v[f + 1, :]
                a2 = a2 + rows_v[f + 2, :]
                a3 = a3 + rows_v[f + 3, :]
            logits = (a0 + a1) + (a2 + a3) + bcv
            pooled_v[s, :] = 1.0 / (1.0 + jnp.exp(-logits))

            @pl.when(g + 1 < NOUTER)
            def _refill():
                pltpu.async_copy(emb2_hbm.at[idx_v.at[s + RING]], bufs[r], sems[r])
        return _

    lax.fori_loop(0, NOUTER, outer_body, None)
    pltpu.sync_copy(pooled_v, out_hbm.at[pl.ds(row_base, BPW)])


_probs_sc = pl.kernel(
    _probs_sc_body,
    out_type=jax.ShapeDtypeStruct((B, D), jnp.float32),
    mesh=_mesh,
    scratch_types=_SC_SCRATCH,
    compiler_params=pltpu.CompilerParams(use_tc_tiling_on_sc=False),
)


def _fold_body(emb_ref, wc_ref, out_ref):
    out_ref[...] = jnp.dot(
        emb_ref[...], wc_ref[...], preferred_element_type=jnp.float32
    )


_FBLK = 4000


def _fold_table(emb, Wcp):
    return pl.pallas_call(
        _fold_body,
        grid=(V // _FBLK,),
        in_specs=[
            pl.BlockSpec((_FBLK, H1), lambda i: (i, 0)),
            pl.BlockSpec((H1, D), lambda i: (0, 0)),
        ],
        out_specs=pl.BlockSpec((_FBLK, D), lambda i: (i, 0)),
        out_shape=jax.ShapeDtypeStruct((V, D), jnp.float32),
    )(emb, Wcp)


def kernel(x, emb, W1, b1, W2, b2, Wout, bout):
    Wc = (W1 @ W2) @ Wout
    bc = (b1 @ W2) @ Wout + b2 @ Wout + bout
    Wcp = jnp.pad(Wc, ((0, 0), (0, D - NCLS)))
    bcv = jnp.pad(bc, (0, D - NCLS))

    emb2 = _fold_table(emb, Wcp)
    probs = _probs_sc(x, emb2, bcv)
    return probs[:, :NCLS]


# speedup vs baseline: 1.3603x; 1.3603x over previous
import jax
import jax.numpy as jnp
from jax import lax
from jax.experimental import pallas as pl
from jax.experimental.pallas import tpu as pltpu
from jax.experimental.pallas import tpu_sc as plsc

B = 16384
F = 100
D = 16
H1, H2, H3 = 16, 64, 16
NCLS = 2

NC = 2
NS = 16
NW = NC * NS
BPW = B // NW
IPW = BPW * F

_mesh = plsc.VectorSubcoreMesh(
    core_axis_name="c", subcore_axis_name="s", num_cores=NC, num_subcores=NS
)


_CHALF = BPW // 2


def _compact_sc_body(x_hbm, xlin_hbm, stage_v, compact_v, perm_v):
    wid = lax.axis_index("s") * NC + lax.axis_index("c")
    row_base = wid * BPW

    lanes = lax.iota(jnp.int32, 16)
    rot4 = lax.bitwise_and(lanes + 4, 15)
    head_mask = lanes < 12
    lo4_mask = lanes < 4
    tail_col = jnp.where(lo4_mask, 96 + lanes, lanes - 4)
    b96_col = jnp.where(head_mask, 0, 84 + lanes)

    for half in range(2):
        pltpu.sync_copy(
            x_hbm.at[pl.ds(row_base + half * _CHALF, _CHALF)], stage_v
        )

        def pair_body(p, _):
            ra = 2 * p
            rb = 2 * p + 1
            base = 200 * p
            ra_v = jnp.full((16,), 0, jnp.int32) + ra
            rb_v = jnp.full((16,), 0, jnp.int32) + rb
            for k in range(6):
                compact_v[pl.ds(base + 16 * k, 16)] = stage_v[ra, pl.ds(16 * k, 16)]
            tail_row = jnp.where(lo4_mask, ra_v, rb_v)
            compact_v[pl.ds(base + 96, 16)] = plsc.load_gather(
                stage_v, [tail_row, tail_col]
            )
            for k in range(6):
                lo = stage_v[rb, pl.ds(16 * k, 16)]
                perm_v[...] = lo
                lo_sh = plsc.load_gather(perm_v, [rot4])
                if k < 5:
                    hi = stage_v[rb, pl.ds(16 * (k + 1), 16)]
                    perm_v[...] = hi
                    hi_sh = plsc.load_gather(perm_v, [rot4])
                else:
                    hi_sh = plsc.load_gather(stage_v, [rb_v, b96_col])
                spl = jnp.where(head_mask, lo_sh, hi_sh)
                compact_v[pl.ds(base + 104 + 16 * k, 16)] = spl
            return _

        lax.fori_loop(0, _CHALF // 2, pair_body, None)
        pltpu.sync_copy(
            compact_v,
            xlin_hbm.at[pl.ds(wid * IPW + half * (_CHALF * F), _CHALF * F)],
        )


_compact_sc = pl.kernel(
    _compact_sc_body,
    out_type=jax.ShapeDtypeStruct((B * F,), jnp.int32),
    mesh=_mesh,
    scratch_types=[
        pltpu.VMEM((_CHALF, F), jnp.int32),
        pltpu.VMEM((_CHALF * F,), jnp.int32),
        pltpu.VMEM((16,), jnp.int32),
    ],
    compiler_params=pltpu.CompilerParams(
        use_tc_tiling_on_sc=True, needs_layout_passes=False
    ),
)


RING = 8
NPAIR = BPW // 2
NOUTER = NPAIR // RING

_SC_SCRATCH = (
    [pltpu.VMEM((IPW,), jnp.int32)]
    + [pltpu.VMEM((2 * F, D), jnp.float32) for _ in range(RING)]
    + [pltpu.VMEM((BPW, D), jnp.float32)]
    + [pltpu.SemaphoreType.DMA for _ in range(RING)]
)


def _pooled_sc_body(xlin_hbm, emb_hbm, out_hbm, idx_v, *rest):
    bufs = rest[:RING]
    pooled_v = rest[RING]
    sems = rest[RING + 1 : 2 * RING + 1]

    wid = lax.axis_index("s") * NC + lax.axis_index("c")
    row_base = wid * BPW

    pltpu.sync_copy(xlin_hbm.at[pl.ds(wid * IPW, IPW)], idx_v)

    for r in range(RING):
        pltpu.async_copy(
            emb_hbm.at[idx_v.at[pl.ds(200 * r, 200)]], bufs[r], sems[r]
        )

    def outer_body(g, _):
        for r in range(RING):
            p = g * RING + r
            pltpu.make_async_copy(
                emb_hbm.at[idx_v.at[pl.ds(200 * p, 200)]], bufs[r], sems[r]
            ).wait()

            rows_v = bufs[r]
            for half in range(2):
                o = half * F
                a0 = rows_v[o + 0, :]
                a1 = rows_v[o + 1, :]
                a2 = rows_v[o + 2, :]
                a3 = rows_v[o + 3, :]
                for f in range(4, F, 4):
                    a0 = a0 + rows_v[o + f + 0, :]
                    a1 = a1 + rows_v[o + f + 1, :]
                    a2 = a2 + rows_v[o + f + 2, :]
                    a3 = a3 + rows_v[o + f + 3, :]
                pooled_v[2 * p + half, :] = (a0 + a1) + (a2 + a3)

            @pl.when(g + 1 < NOUTER)
            def _refill():
                pltpu.async_copy(
                    emb_hbm.at[idx_v.at[pl.ds(200 * (p + RING), 200)]],
                    bufs[r],
                    sems[r],
                )
        return _

    lax.fori_loop(0, NOUTER, outer_body, None)
    pltpu.sync_copy(pooled_v, out_hbm.at[pl.ds(row_base, BPW)])


_pooled_sc = pl.kernel(
    _pooled_sc_body,
    out_type=jax.ShapeDtypeStruct((B, D), jnp.float32),
    mesh=_mesh,
    scratch_types=_SC_SCRATCH,
    compiler_params=pltpu.CompilerParams(use_tc_tiling_on_sc=False),
)


def _head_body(p_ref, w1_ref, b1_ref, w2_ref, b2_ref, wo_ref, bo_ref, o_ref):
    h = p_ref[...]
    h = jnp.dot(h, w1_ref[...], preferred_element_type=jnp.float32) + b1_ref[...]
    h = jnp.dot(h, w2_ref[...], preferred_element_type=jnp.float32) + b2_ref[...]
    h = jnp.dot(h, wo_ref[...], preferred_element_type=jnp.float32) + bo_ref[...]
    o_ref[...] = jax.nn.sigmoid(h)


_HBLK = 4096


def _head(pooled, W1, b1, W2, b2, Wout, bout):
    grid = (B // _HBLK,)
    return pl.pallas_call(
        _head_body,
        grid=grid,
        in_specs=[
            pl.BlockSpec((_HBLK, D), lambda i: (i, 0)),
            pl.BlockSpec((H1, H2), lambda i: (0, 0)),
            pl.BlockSpec((1, H2), lambda i: (0, 0)),
            pl.BlockSpec((H2, H3), lambda i: (0, 0)),
            pl.BlockSpec((1, H3), lambda i: (0, 0)),
            pl.BlockSpec((H3, NCLS), lambda i: (0, 0)),
            pl.BlockSpec((1, NCLS), lambda i: (0, 0)),
        ],
        out_specs=pl.BlockSpec((_HBLK, NCLS), lambda i: (i, 0)),
        out_shape=jax.ShapeDtypeStruct((B, NCLS), jnp.float32),
    )(
        pooled,
        W1,
        b1.reshape(1, H2),
        W2,
        b2.reshape(1, H3),
        Wout,
        bout.reshape(1, NCLS),
    )


def kernel(x, emb, W1, b1, W2, b2, Wout, bout):
    xlin = _compact_sc(x)
    pooled = _pooled_sc(xlin, emb)
    return _head(pooled, W1, b1, W2, b2, Wout, bout)


# speedup vs baseline: 1.4148x; 1.0400x over previous
import jax
import jax.numpy as jnp
from jax import lax
from jax.experimental import pallas as pl
from jax.experimental.pallas import tpu as pltpu
from jax.experimental.pallas import tpu_sc as plsc

B = 16384
F = 100
D = 16
H1, H2, H3 = 16, 64, 16
NCLS = 2

NC = 2
NS = 16
NW = NC * NS
BPW = B // NW
IPW = BPW * F

_mesh = plsc.VectorSubcoreMesh(
    core_axis_name="c", subcore_axis_name="s", num_cores=NC, num_subcores=NS
)


_CHALF = BPW // 2


def _compact_sc_body(x_hbm, xlin_hbm, stage_v, compact_v):
    wid = lax.axis_index("s") * NC + lax.axis_index("c")
    row_base = wid * BPW

    lanes = lax.iota(jnp.int32, 16)
    lo4_mask = lanes < 4
    tail_col = jnp.where(lo4_mask, 96 + lanes, lanes - 4)
    b_cols = [lanes + (4 + 16 * k) for k in range(5)] + [lanes + 84]

    for half in range(2):
        pltpu.sync_copy(
            x_hbm.at[pl.ds(row_base + half * _CHALF, _CHALF)], stage_v
        )

        def pair_body(p, _):
            ra = 2 * p
            rb = 2 * p + 1
            base = 200 * p
            ra_v = jnp.zeros((16,), jnp.int32) + ra
            rb_v = jnp.zeros((16,), jnp.int32) + rb
            for k in range(6):
                compact_v[pl.ds(base + 16 * k, 16)] = stage_v[ra, pl.ds(16 * k, 16)]
            tail_row = jnp.where(lo4_mask, ra_v, rb_v)
            compact_v[pl.ds(base + 96, 16)] = plsc.load_gather(
                stage_v, [tail_row, tail_col]
            )
            for k in range(6):
                compact_v[pl.ds(base + 104 + 16 * k, 16)] = plsc.load_gather(
                    stage_v, [rb_v, b_cols[k]]
                )
            return _

        lax.fori_loop(0, _CHALF // 2, pair_body, None)
        pltpu.sync_copy(
            compact_v,
            xlin_hbm.at[pl.ds(wid * IPW + half * (_CHALF * F), _CHALF * F)],
        )


_compact_sc = pl.kernel(
    _compact_sc_body,
    out_type=jax.ShapeDtypeStruct((B * F,), jnp.int32),
    mesh=_mesh,
    scratch_types=[
        pltpu.VMEM((_CHALF, F), jnp.int32),
        pltpu.VMEM((_CHALF * F,), jnp.int32),
    ],
    compiler_params=pltpu.CompilerParams(
        use_tc_tiling_on_sc=True, needs_layout_passes=False
    ),
)


RING = 8
NPAIR = BPW // 2
NOUTER = NPAIR // RING

_SC_SCRATCH = (
    [pltpu.VMEM((IPW,), jnp.int32)]
    + [pltpu.VMEM((2 * F, D), jnp.float32) for _ in range(RING)]
    + [pltpu.VMEM((BPW, D), jnp.float32)]
    + [pltpu.SemaphoreType.DMA for _ in range(RING)]
)


def _pooled_sc_body(xlin_hbm, emb_hbm, out_hbm, idx_v, *rest):
    bufs = rest[:RING]
    pooled_v = rest[RING]
    sems = rest[RING + 1 : 2 * RING + 1]

    wid = lax.axis_index("s") * NC + lax.axis_index("c")
    row_base = wid * BPW

    pltpu.sync_copy(xlin_hbm.at[pl.ds(wid * IPW, IPW)], idx_v)

    for r in range(RING):
        pltpu.async_copy(
            emb_hbm.at[idx_v.at[pl.ds(200 * r, 200)]], bufs[r], sems[r]
        )

    def outer_body(g, _):
        for r in range(RING):
            p = g * RING + r
            pltpu.make_async_copy(
                emb_hbm.at[idx_v.at[pl.ds(200 * p, 200)]], bufs[r], sems[r]
            ).wait()

            rows_v = bufs[r]
            for half in range(2):
                o = half * F
                a0 = rows_v[o + 0, :]
                a1 = rows_v[o + 1, :]
                a2 = rows_v[o + 2, :]
                a3 = rows_v[o + 3, :]
                for f in range(4, F, 4):
                    a0 = a0 + rows_v[o + f + 0, :]
                    a1 = a1 + rows_v[o + f + 1, :]
                    a2 = a2 + rows_v[o + f + 2, :]
                    a3 = a3 + rows_v[o + f + 3, :]
                pooled_v[2 * p + half, :] = (a0 + a1) + (a2 + a3)

            @pl.when(g + 1 < NOUTER)
            def _refill():
                pltpu.async_copy(
                    emb_hbm.at[idx_v.at[pl.ds(200 * (p + RING), 200)]],
                    bufs[r],
                    sems[r],
                )
        return _

    lax.fori_loop(0, NOUTER, outer_body, None)
    pltpu.sync_copy(pooled_v, out_hbm.at[pl.ds(row_base, BPW)])


_pooled_sc = pl.kernel(
    _pooled_sc_body,
    out_type=jax.ShapeDtypeStruct((B, D), jnp.float32),
    mesh=_mesh,
    scratch_types=_SC_SCRATCH,
    compiler_params=pltpu.CompilerParams(use_tc_tiling_on_sc=False),
)


def _head_body(p_ref, w1_ref, b1_ref, w2_ref, b2_ref, wo_ref, bo_ref, o_ref):
    h = p_ref[...]
    h = jnp.dot(h, w1_ref[...], preferred_element_type=jnp.float32) + b1_ref[...]
    h = jnp.dot(h, w2_ref[...], preferred_element_type=jnp.float32) + b2_ref[...]
    h = jnp.dot(h, wo_ref[...], preferred_element_type=jnp.float32) + bo_ref[...]
    o_ref[...] = jax.nn.sigmoid(h)


_HBLK = 4096


def _head(pooled, W1, b1, W2, b2, Wout, bout):
    grid = (B // _HBLK,)
    return pl.pallas_call(
        _head_body,
        grid=grid,
        in_specs=[
            pl.BlockSpec((_HBLK, D), lambda i: (i, 0)),
            pl.BlockSpec((H1, H2), lambda i: (0, 0)),
            pl.BlockSpec((1, H2), lambda i: (0, 0)),
            pl.BlockSpec((H2, H3), lambda i: (0, 0)),
            pl.BlockSpec((1, H3), lambda i: (0, 0)),
            pl.BlockSpec((H3, NCLS), lambda i: (0, 0)),
            pl.BlockSpec((1, NCLS), lambda i: (0, 0)),
        ],
        out_specs=pl.BlockSpec((_HBLK, NCLS), lambda i: (i, 0)),
        out_shape=jax.ShapeDtypeStruct((B, NCLS), jnp.float32),
    )(
        pooled,
        W1,
        b1.reshape(1, H2),
        W2,
        b2.reshape(1, H3),
        Wout,
        bout.reshape(1, NCLS),
    )


def kernel(x, emb, W1, b1, W2, b2, Wout, bout):
    xlin = _compact_sc(x)
    pooled = _pooled_sc(xlin, emb)
    return _head(pooled, W1, b1, W2, b2, Wout, bout)


# speedup vs baseline: 1.4431x; 1.0200x over previous
import jax
import jax.numpy as jnp
from jax import lax
from jax.experimental import pallas as pl
from jax.experimental.pallas import tpu as pltpu
from jax.experimental.pallas import tpu_sc as plsc

B = 16384
F = 100
D = 16
H1, H2, H3 = 16, 64, 16
NCLS = 2

NC = 2
NS = 16
NW = NC * NS
BPW = B // NW
IPW = BPW * F

_mesh = plsc.VectorSubcoreMesh(
    core_axis_name="c", subcore_axis_name="s", num_cores=NC, num_subcores=NS
)


_CHALF = BPW // 2


def _compact_sc_body(x_hbm, xlin_hbm, stage_v, compact_v):
    wid = lax.axis_index("s") * NC + lax.axis_index("c")
    row_base = wid * BPW

    lanes = lax.iota(jnp.int32, 16)
    lo4_mask = lanes < 4
    tail_col = jnp.where(lo4_mask, 96 + lanes, lanes - 4)
    b_cols = [lanes + (4 + 16 * k) for k in range(5)] + [lanes + 84]

    for half in range(2):
        pltpu.sync_copy(
            x_hbm.at[pl.ds(row_base + half * _CHALF, _CHALF)], stage_v
        )

        def pair_body(p, _):
            ra = 2 * p
            rb = 2 * p + 1
            base = 200 * p
            ra_v = jnp.zeros((16,), jnp.int32) + ra
            rb_v = jnp.zeros((16,), jnp.int32) + rb
            for k in range(6):
                compact_v[pl.ds(base + 16 * k, 16)] = stage_v[ra, pl.ds(16 * k, 16)]
            tail_row = jnp.where(lo4_mask, ra_v, rb_v)
            compact_v[pl.ds(base + 96, 16)] = plsc.load_gather(
                stage_v, [tail_row, tail_col]
            )
            for k in range(6):
                compact_v[pl.ds(base + 104 + 16 * k, 16)] = plsc.load_gather(
                    stage_v, [rb_v, b_cols[k]]
                )
            return _

        lax.fori_loop(0, _CHALF // 2, pair_body, None)
        pltpu.sync_copy(
            compact_v,
            xlin_hbm.at[pl.ds(wid * IPW + half * (_CHALF * F), _CHALF * F)],
        )


_compact_sc = pl.kernel(
    _compact_sc_body,
    out_type=jax.ShapeDtypeStruct((B * F,), jnp.int32),
    mesh=_mesh,
    scratch_types=[
        pltpu.VMEM((_CHALF, F), jnp.int32),
        pltpu.VMEM((_CHALF * F,), jnp.int32),
    ],
    compiler_params=pltpu.CompilerParams(
        use_tc_tiling_on_sc=True, needs_layout_passes=False
    ),
)


RING = 8
NPAIR = BPW // 2
NOUTER = NPAIR // RING

_POOL_ROWS = BPW * D // 128

_SC_SCRATCH = (
    [pltpu.VMEM((IPW,), jnp.int32)]
    + [pltpu.VMEM((2 * F, D), jnp.float32) for _ in range(RING)]
    + [pltpu.VMEM((_POOL_ROWS, 128), jnp.float32)]
    + [pltpu.SemaphoreType.DMA for _ in range(RING)]
)


def _pooled_sc_body(xlin_hbm, emb_hbm, out_hbm, idx_v, *rest):
    bufs = rest[:RING]
    pooled_v = rest[RING]
    sems = rest[RING + 1 : 2 * RING + 1]

    wid = lax.axis_index("s") * NC + lax.axis_index("c")
    row_base = wid * BPW

    pltpu.sync_copy(xlin_hbm.at[pl.ds(wid * IPW, IPW)], idx_v)

    for r in range(RING):
        pltpu.async_copy(
            emb_hbm.at[idx_v.at[pl.ds(200 * r, 200)]], bufs[r], sems[r]
        )

    def outer_body(g, _):
        for r in range(RING):
            p = g * RING + r
            pltpu.make_async_copy(
                emb_hbm.at[idx_v.at[pl.ds(200 * p, 200)]], bufs[r], sems[r]
            ).wait()

            rows_v = bufs[r]
            for half in range(2):
                o = half * F
                a0 = rows_v[o + 0, :]
                a1 = rows_v[o + 1, :]
                a2 = rows_v[o + 2, :]
                a3 = rows_v[o + 3, :]
                for f in range(4, F, 4):
                    a0 = a0 + rows_v[o + f + 0, :]
                    a1 = a1 + rows_v[o + f + 1, :]
                    a2 = a2 + rows_v[o + f + 2, :]
                    a3 = a3 + rows_v[o + f + 3, :]
                s = 2 * p + half
                pooled_v[
                    lax.shift_right_logical(s, 3),
                    pl.ds(lax.bitwise_and(s, 7) * D, D),
                ] = (a0 + a1) + (a2 + a3)

            @pl.when(g + 1 < NOUTER)
            def _refill():
                pltpu.async_copy(
                    emb_hbm.at[idx_v.at[pl.ds(200 * (p + RING), 200)]],
                    bufs[r],
                    sems[r],
                )
        return _

    lax.fori_loop(0, NOUTER, outer_body, None)
    pltpu.sync_copy(pooled_v, out_hbm.at[pl.ds(wid * _POOL_ROWS, _POOL_ROWS)])


_pooled_sc = pl.kernel(
    _pooled_sc_body,
    out_type=jax.ShapeDtypeStruct((B * D // 128, 128), jnp.float32),
    mesh=_mesh,
    scratch_types=_SC_SCRATCH,
    compiler_params=pltpu.CompilerParams(use_tc_tiling_on_sc=False),
)


def _head_body(p_ref, w1_ref, b1_ref, w2_ref, b2_ref, wo_ref, bo_ref, o_ref):
    h = p_ref[...]
    h = jnp.dot(h, w1_ref[...], preferred_element_type=jnp.float32) + b1_ref[...]
    h = jnp.dot(h, w2_ref[...], preferred_element_type=jnp.float32) + b2_ref[...]
    h = jnp.dot(h, wo_ref[...], preferred_element_type=jnp.float32) + bo_ref[...]
    o_ref[...] = jax.nn.sigmoid(h)


_HBLK = 4096
_HROWS = _HBLK * D // 128
_PK = 128 // D


def _head(pooled, W1, b1, W2, b2, Wout, bout):
    eye = jnp.eye(_PK, dtype=jnp.float32)
    W1b = jnp.kron(eye, W1)
    W2b = jnp.kron(eye, W2)
    Wob = jnp.kron(eye, Wout)
    b1b = jnp.tile(b1, _PK).reshape(1, _PK * H2)
    b2b = jnp.tile(b2, _PK).reshape(1, _PK * H3)
    bob = jnp.tile(bout, _PK).reshape(1, _PK * NCLS)
    grid = (B // _HBLK,)
    return pl.pallas_call(
        _head_body,
        grid=grid,
        in_specs=[
            pl.BlockSpec((_HROWS, 128), lambda i: (i, 0)),
            pl.BlockSpec((128, _PK * H2), lambda i: (0, 0)),
            pl.BlockSpec((1, _PK * H2), lambda i: (0, 0)),
            pl.BlockSpec((_PK * H2, _PK * H3), lambda i: (0, 0)),
            pl.BlockSpec((1, _PK * H3), lambda i: (0, 0)),
            pl.BlockSpec((_PK * H3, _PK * NCLS), lambda i: (0, 0)),
            pl.BlockSpec((1, _PK * NCLS), lambda i: (0, 0)),
        ],
        out_specs=pl.BlockSpec((_HROWS, _PK * NCLS), lambda i: (i, 0)),
        out_shape=jax.ShapeDtypeStruct((B * D // 128, _PK * NCLS), jnp.float32),
    )(pooled, W1b, b1b, W2b, b2b, Wob, bob)


def kernel(x, emb, W1, b1, W2, b2, Wout, bout):
    xlin = _compact_sc(x)
    pooled = _pooled_sc(xlin, emb)
    packed = _head(pooled, W1, b1, W2, b2, Wout, bout)
    return packed.reshape(B, NCLS)
